# baseline (speedup 1.0000x reference)
import functools

import jax
import jax.numpy as jnp
from jax import lax
from jax.experimental import pallas as pl
from jax.experimental.pallas import tpu as pltpu
from jax.experimental.pallas import tpu_sc as plsc

N = 10000
E = 320000
D = 128
G = 64
L = 3

NC = 2
NS = 16
NW = NC * NS
NPAD = 10240
RPS = NPAD // NS
WIN = 128
NWINP = 2560
EPAD = NWINP * WIN
WPW = NWINP // NW
_mesh = plsc.VectorSubcoreMesh(core_axis_name="c", subcore_axis_name="s")


@functools.partial(
    pl.kernel,
    out_type=jax.ShapeDtypeStruct((NC, NPAD, D), jnp.float32),
    mesh=_mesh,
    scratch_types=[
        pltpu.VMEM_SHARED((NPAD, D), jnp.float32),
        [pltpu.VMEM((2, WIN), jnp.int32)] * 2,
        [pltpu.VMEM((WIN, D), jnp.float32)] * 2,
        pltpu.SemaphoreType.DMA,
    ],
)
def _sc_agg_kernel(h_hbm, e_hbm, z_hbm, out_hbm, acc, xb, rb, sem):
    c = lax.axis_index("c")
    s = lax.axis_index("s")
    wid = s * NC + c
    base = wid * WPW

    pltpu.sync_copy(z_hbm, acc.at[pl.ds(s * RPS, RPS)])
    plsc.subcore_barrier()

    pltpu.sync_copy(e_hbm.at[base], xb[0])
    pltpu.sync_copy(h_hbm.at[xb[0].at[0]], rb[0])

    @pl.loop(0, WPW, step=2)
    def _(vb):
        for u in range(2):
            v = vb + u
            p = u & 1

            @pl.when(v + 1 < WPW)
            def _():
                pltpu.sync_copy(e_hbm.at[base + v + 1], xb[p ^ 1])
                pltpu.sync_copy(h_hbm.at[xb[p ^ 1].at[0]], rb[p ^ 1])

            pltpu.sync_copy(rb[p], acc.at[xb[p].at[1]], add=True)

    plsc.subcore_barrier()
    pltpu.sync_copy(acc.at[pl.ds(s * RPS, RPS)],
                    out_hbm.at[c, pl.ds(s * RPS, RPS)])


def _sc_agg(h, edge_index3, zeros):
    return _sc_agg_kernel(h, edge_index3, zeros)


def _dot_t(a, w):
    return lax.dot_general(a, w, (((1,), (1,)), ((), ())),
                           preferred_element_type=jnp.float32)


def _tc_init_body(x_ref, w_ref, b_ref, o_ref):
    o_ref[...] = _dot_t(x_ref[...], w_ref[...]) + b_ref[...]


def _tc_init(x, W_init, b2):
    return pl.pallas_call(
        _tc_init_body,
        out_shape=jax.ShapeDtypeStruct((N, D), jnp.float32),
    )(x, W_init, b2)


def _tc_layer_body(h_ref, p_ref, wr_ref, br_ref, wt_ref, g_ref, b_ref, o_ref):
    agg = p_ref[0, :N, :] + p_ref[1, :N, :]
    h = h_ref[...]
    t = h + _dot_t(agg, wr_ref[...]) + br_ref[...] + _dot_t(h, wt_ref[...])
    m = jnp.mean(t, axis=0, keepdims=True)
    v = jnp.mean((t - m) ** 2, axis=0, keepdims=True)
    t = (t - m) / jnp.sqrt(v + 1e-5) * g_ref[...] + b_ref[...]
    o_ref[...] = jnp.maximum(t, 0.0)


def _tc_layer(h, parts, Wr, br2, Wt, g2, b2):
    return pl.pallas_call(
        _tc_layer_body,
        out_shape=jax.ShapeDtypeStruct((N, D), jnp.float32),
    )(h, parts, Wr, br2, Wt, g2, b2)


def _tc_final_body(h_ref, p_ref, wr_ref, br_ref, wt_ref, batch_ref, o_ref):
    agg = p_ref[0, :N, :] + p_ref[1, :N, :]
    t = _dot_t(agg, wr_ref[...]) + br_ref[...] + _dot_t(h_ref[...], wt_ref[...])
    seg = lax.broadcasted_iota(jnp.int32, (G, N), 0)
    mask = (seg == batch_ref[...]).astype(jnp.float32)
    o_ref[...] = lax.dot_general(mask, t, (((1,), (0,)), ((), ())),
                                 preferred_element_type=jnp.float32)


def _tc_final(h, parts, Wr, br2, Wt, batch2):
    return pl.pallas_call(
        _tc_final_body,
        out_shape=jax.ShapeDtypeStruct((G, D), jnp.float32),
    )(h, parts, Wr, br2, Wt, batch2)


def kernel(x, edge_index, batch, W_init, b_init, W_rel, b_rel, W_root, gamma, beta):
    zeros = jnp.zeros((RPS, D), jnp.float32)
    batch2 = batch.reshape(1, N)
    pad = jnp.stack([jnp.zeros((EPAD - E,), jnp.int32),
                     jnp.full((EPAD - E,), NPAD - 1, jnp.int32)])
    e3 = jnp.concatenate([edge_index, pad], axis=1) \
            .reshape(2, NWINP, WIN).transpose(1, 0, 2)
    h = _tc_init(x, W_init, b_init.reshape(1, D))
    for i in range(L - 1):
        parts = _sc_agg(h, e3, zeros)
        h = _tc_layer(h, parts, W_rel[i], b_rel[i].reshape(1, D),
                      W_root[i], gamma[i].reshape(1, D), beta[i].reshape(1, D))
    parts = _sc_agg(h, e3, zeros)
    return _tc_final(h, parts, W_rel[L - 1], b_rel[L - 1].reshape(1, D),
                     W_root[L - 1], batch2)


# speedup vs baseline: 2.5946x; 2.5946x over previous
import functools

import jax
import jax.numpy as jnp
from jax import lax
from jax.experimental import pallas as pl
from jax.experimental.pallas import tpu as pltpu
from jax.experimental.pallas import tpu_sc as plsc

N = 10000
E = 320000
D = 128
G = 64
L = 3

NC = 2
NS = 16
NW = NC * NS
NPAD = 10240
RPS = NPAD // NS
WIN = 128
NWIN = E // WIN
WPW = -(-NWIN // NW)
_mesh = plsc.VectorSubcoreMesh(core_axis_name="c", subcore_axis_name="s")


@functools.partial(
    pl.kernel,
    out_type=jax.ShapeDtypeStruct((NC, NPAD, D), jnp.float32),
    mesh=_mesh,
    scratch_types=[
        pltpu.VMEM_SHARED((NPAD, D), jnp.float32),
        [pltpu.VMEM((WIN,), jnp.int32)] * 2,
        [pltpu.VMEM((WIN,), jnp.int32)] * 2,
        [pltpu.VMEM((WIN, D), jnp.float32)] * 2,
        pltpu.SemaphoreType.DMA,
    ],
)
def _sc_agg_kernel(h_hbm, e_hbm, z_hbm, out_hbm, acc, sb, db, rb, sem):
    c = lax.axis_index("c")
    s = lax.axis_index("s")
    wid = s * NC + c

    pltpu.sync_copy(z_hbm, acc.at[pl.ds(s * RPS, RPS)])
    plsc.subcore_barrier()

    pltpu.sync_copy(e_hbm.at[0, pl.ds(wid * WIN, WIN)], sb[0])
    pltpu.sync_copy(e_hbm.at[1, pl.ds(wid * WIN, WIN)], db[0])
    pltpu.sync_copy(h_hbm.at[sb[0]], rb[0])

    @pl.loop(0, WPW + 1, step=2)
    def _(vb):
        for u in range(2):
            v = vb + u
            p = u & 1
            w = wid + NW * v
            desc = None

            @pl.when(w + NW < NWIN)
            def _():
                nonlocal desc
                off = (w + NW) * WIN
                pltpu.sync_copy(e_hbm.at[0, pl.ds(off, WIN)], sb[p ^ 1])
                pltpu.sync_copy(e_hbm.at[1, pl.ds(off, WIN)], db[p ^ 1])
                desc = pltpu.async_copy(h_hbm.at[sb[p ^ 1]], rb[p ^ 1], sem)

            @pl.when(w < NWIN)
            def _():
                pltpu.sync_copy(rb[p], acc.at[db[p]], add=True)

            @pl.when(w + NW < NWIN)
            def _():
                desc.wait()

    plsc.subcore_barrier()
    pltpu.sync_copy(acc.at[pl.ds(s * RPS, RPS)],
                    out_hbm.at[c, pl.ds(s * RPS, RPS)])


def _sc_agg(h, edge_index3, zeros):
    return _sc_agg_kernel(h, edge_index3, zeros)


def _dot_t(a, w):
    return lax.dot_general(a, w, (((1,), (1,)), ((), ())),
                           preferred_element_type=jnp.float32)


def _tc_init_body(x_ref, w_ref, b_ref, o_ref):
    o_ref[...] = _dot_t(x_ref[...], w_ref[...]) + b_ref[...]


def _tc_init(x, W_init, b2):
    return pl.pallas_call(
        _tc_init_body,
        out_shape=jax.ShapeDtypeStruct((N, D), jnp.float32),
    )(x, W_init, b2)


def _tc_layer_body(h_ref, p_ref, wr_ref, br_ref, wt_ref, g_ref, b_ref, o_ref):
    agg = p_ref[0, :N, :] + p_ref[1, :N, :]
    h = h_ref[...]
    t = h + _dot_t(agg, wr_ref[...]) + br_ref[...] + _dot_t(h, wt_ref[...])
    m = jnp.mean(t, axis=0, keepdims=True)
    v = jnp.mean((t - m) ** 2, axis=0, keepdims=True)
    t = (t - m) / jnp.sqrt(v + 1e-5) * g_ref[...] + b_ref[...]
    o_ref[...] = jnp.maximum(t, 0.0)


def _tc_layer(h, parts, Wr, br2, Wt, g2, b2):
    return pl.pallas_call(
        _tc_layer_body,
        out_shape=jax.ShapeDtypeStruct((N, D), jnp.float32),
    )(h, parts, Wr, br2, Wt, g2, b2)


def _tc_final_body(h_ref, p_ref, wr_ref, br_ref, wt_ref, batch_ref, o_ref):
    agg = p_ref[0, :N, :] + p_ref[1, :N, :]
    t = _dot_t(agg, wr_ref[...]) + br_ref[...] + _dot_t(h_ref[...], wt_ref[...])
    seg = lax.broadcasted_iota(jnp.int32, (G, N), 0)
    mask = (seg == batch_ref[...]).astype(jnp.float32)
    o_ref[...] = lax.dot_general(mask, t, (((1,), (0,)), ((), ())),
                                 preferred_element_type=jnp.float32)


def _tc_final(h, parts, Wr, br2, Wt, batch2):
    return pl.pallas_call(
        _tc_final_body,
        out_shape=jax.ShapeDtypeStruct((G, D), jnp.float32),
    )(h, parts, Wr, br2, Wt, batch2)


def kernel(x, edge_index, batch, W_init, b_init, W_rel, b_rel, W_root, gamma, beta):
    zeros = jnp.zeros((RPS, D), jnp.float32)
    batch2 = batch.reshape(1, N)
    e3 = edge_index
    h = _tc_init(x, W_init, b_init.reshape(1, D))
    for i in range(L - 1):
        parts = _sc_agg(h, e3, zeros)
        h = _tc_layer(h, parts, W_rel[i], b_rel[i].reshape(1, D),
                      W_root[i], gamma[i].reshape(1, D), beta[i].reshape(1, D))
    parts = _sc_agg(h, e3, zeros)
    return _tc_final(h, parts, W_rel[L - 1], b_rel[L - 1].reshape(1, D),
                     W_root[L - 1], batch2)


# speedup vs baseline: 3.7904x; 1.4609x over previous
import functools

import jax
import jax.numpy as jnp
from jax import lax
from jax.experimental import pallas as pl
from jax.experimental.pallas import tpu as pltpu
from jax.experimental.pallas import tpu_sc as plsc

N = 10000
E = 320000
D = 128
G = 64
L = 3

NC = 2
NS = 16
NW = NC * NS
NPAD = 10240
RPS = NPAD // NS
WIN = 128
NWIN = E // WIN
WPW = -(-NWIN // NW)
_mesh = plsc.VectorSubcoreMesh(core_axis_name="c", subcore_axis_name="s")


@functools.partial(
    pl.kernel,
    out_type=jax.ShapeDtypeStruct((NC, NPAD, D), jnp.float32),
    mesh=_mesh,
    scratch_types=[
        pltpu.VMEM_SHARED((NPAD, D), jnp.float32),
        [pltpu.VMEM((WIN,), jnp.int32)] * 4,
        [pltpu.VMEM((WIN,), jnp.int32)] * 4,
        [pltpu.VMEM((WIN, D), jnp.float32)] * 2,
        pltpu.SemaphoreType.DMA,
        pltpu.SemaphoreType.DMA,
    ],
)
def _sc_agg_kernel(h_hbm, e_hbm, z_hbm, out_hbm, acc, sb, db, rb, sem, semi):
    c = lax.axis_index("c")
    s = lax.axis_index("s")
    wid = s * NC + c

    pltpu.sync_copy(z_hbm, acc.at[pl.ds(s * RPS, RPS)])
    plsc.subcore_barrier()

    def idx_fire(v, sl):
        off = (wid + NW * v) * WIN
        return (pltpu.async_copy(e_hbm.at[0, pl.ds(off, WIN)], sb[sl], semi),
                pltpu.async_copy(e_hbm.at[1, pl.ds(off, WIN)], db[sl], semi))

    for d in idx_fire(0, 0):
        d.wait()
    for d in idx_fire(1, 1):
        d.wait()
    pltpu.sync_copy(h_hbm.at[sb[0]], rb[0])

    @pl.loop(0, WPW + 1, step=4)
    def _(vb):
        for u in range(4):
            v = vb + u
            p = u & 1
            sl = u & 3
            w = wid + NW * v
            off2 = (wid + NW * (v + 2)) * WIN
            s_view = e_hbm.at[0, pl.ds(off2, WIN)]
            d_view = e_hbm.at[1, pl.ds(off2, WIN)]
            idescs = ()
            gdesc = None

            @pl.when(w + 2 * NW < NWIN)
            def _():
                nonlocal idescs
                sl2 = (u + 2) & 3
                idescs = (pltpu.async_copy(s_view, sb[sl2], semi),
                          pltpu.async_copy(d_view, db[sl2], semi))

            @pl.when(w + NW < NWIN)
            def _():
                nonlocal gdesc
                gdesc = pltpu.async_copy(h_hbm.at[sb[(u + 1) & 3]],
                                         rb[p ^ 1], sem)

            @pl.when(w < NWIN)
            def _():
                pltpu.sync_copy(rb[p], acc.at[db[sl]], add=True)

            @pl.when(w + NW < NWIN)
            def _():
                gdesc.wait()

            @pl.when(w + 2 * NW < NWIN)
            def _():
                for d in idescs:
                    d.wait()

    plsc.subcore_barrier()
    pltpu.sync_copy(acc.at[pl.ds(s * RPS, RPS)],
                    out_hbm.at[c, pl.ds(s * RPS, RPS)])


def _sc_agg(h, edge_index3, zeros):
    return _sc_agg_kernel(h, edge_index3, zeros)


def _dot_t(a, w):
    return lax.dot_general(a, w, (((1,), (1,)), ((), ())),
                           preferred_element_type=jnp.float32)


def _tc_init_body(x_ref, w_ref, b_ref, o_ref):
    o_ref[...] = _dot_t(x_ref[...], w_ref[...]) + b_ref[...]


def _tc_init(x, W_init, b2):
    return pl.pallas_call(
        _tc_init_body,
        out_shape=jax.ShapeDtypeStruct((N, D), jnp.float32),
    )(x, W_init, b2)


def _tc_layer_body(h_ref, p_ref, wr_ref, br_ref, wt_ref, g_ref, b_ref, o_ref):
    agg = p_ref[0, :N, :] + p_ref[1, :N, :]
    h = h_ref[...]
    t = h + _dot_t(agg, wr_ref[...]) + br_ref[...] + _dot_t(h, wt_ref[...])
    m = jnp.mean(t, axis=0, keepdims=True)
    v = jnp.mean((t - m) ** 2, axis=0, keepdims=True)
    t = (t - m) / jnp.sqrt(v + 1e-5) * g_ref[...] + b_ref[...]
    o_ref[...] = jnp.maximum(t, 0.0)


def _tc_layer(h, parts, Wr, br2, Wt, g2, b2):
    return pl.pallas_call(
        _tc_layer_body,
        out_shape=jax.ShapeDtypeStruct((N, D), jnp.float32),
    )(h, parts, Wr, br2, Wt, g2, b2)


def _tc_final_body(h_ref, p_ref, wr_ref, br_ref, wt_ref, batch_ref, o_ref):
    agg = p_ref[0, :N, :] + p_ref[1, :N, :]
    t = _dot_t(agg, wr_ref[...]) + br_ref[...] + _dot_t(h_ref[...], wt_ref[...])
    seg = lax.broadcasted_iota(jnp.int32, (G, N), 0)
    mask = (seg == batch_ref[...]).astype(jnp.float32)
    o_ref[...] = lax.dot_general(mask, t, (((1,), (0,)), ((), ())),
                                 preferred_element_type=jnp.float32)


def _tc_final(h, parts, Wr, br2, Wt, batch2):
    return pl.pallas_call(
        _tc_final_body,
        out_shape=jax.ShapeDtypeStruct((G, D), jnp.float32),
    )(h, parts, Wr, br2, Wt, batch2)


def kernel(x, edge_index, batch, W_init, b_init, W_rel, b_rel, W_root, gamma, beta):
    zeros = jnp.zeros((RPS, D), jnp.float32)
    batch2 = batch.reshape(1, N)
    e3 = edge_index
    h = _tc_init(x, W_init, b_init.reshape(1, D))
    for i in range(L - 1):
        parts = _sc_agg(h, e3, zeros)
        h = _tc_layer(h, parts, W_rel[i], b_rel[i].reshape(1, D),
                      W_root[i], gamma[i].reshape(1, D), beta[i].reshape(1, D))
    parts = _sc_agg(h, e3, zeros)
    return _tc_final(h, parts, W_rel[L - 1], b_rel[L - 1].reshape(1, D),
                     W_root[L - 1], batch2)


# speedup vs baseline: 4.9699x; 1.3112x over previous
import functools

import jax
import jax.numpy as jnp
from jax import lax
from jax.experimental import pallas as pl
from jax.experimental.pallas import tpu as pltpu
from jax.experimental.pallas import tpu_sc as plsc

N = 10000
E = 320000
D = 128
G = 64
L = 3

NC = 2
NS = 16
NW = NC * NS
NPAD = 10000
RPS = 632
RPSL = NPAD - RPS * (NS - 1)
WIN = 128
NWIN = E // WIN
WPW = -(-NWIN // NW)
_mesh = plsc.VectorSubcoreMesh(core_axis_name="c", subcore_axis_name="s")


@functools.partial(
    pl.kernel,
    out_type=jax.ShapeDtypeStruct((NC, NPAD, D), jnp.float32),
    mesh=_mesh,
    scratch_types=[
        pltpu.VMEM_SHARED((NPAD, D), jnp.float32),
        [pltpu.VMEM((WIN,), jnp.int32)] * 4,
        [pltpu.VMEM((WIN,), jnp.int32)] * 4,
        [pltpu.VMEM((WIN, D), jnp.float32)] * 3,
        pltpu.SemaphoreType.DMA,
        pltpu.SemaphoreType.DMA,
    ],
)
def _sc_agg_kernel(h_hbm, e_hbm, z_hbm, out_hbm, acc, sb, db, rb, sem, semi):
    c = lax.axis_index("c")
    s = lax.axis_index("s")
    wid = s * NC + c

    @pl.when(s < NS - 1)
    def _():
        pltpu.sync_copy(z_hbm, acc.at[pl.ds(s * RPS, RPS)])

    @pl.when(s == NS - 1)
    def _():
        pltpu.sync_copy(z_hbm.at[pl.ds(0, RPSL)],
                        acc.at[pl.ds((NS - 1) * RPS, RPSL)])

    plsc.subcore_barrier()

    def idx_fire(v, sl):
        off = (wid + NW * v) * WIN
        return (pltpu.async_copy(e_hbm.at[0, pl.ds(off, WIN)], sb[sl], semi),
                pltpu.async_copy(e_hbm.at[1, pl.ds(off, WIN)], db[sl], semi))

    def idx_wait(sl):
        pltpu.make_async_copy(e_hbm.at[0, pl.ds(0, WIN)], sb[sl], semi).wait()
        pltpu.make_async_copy(e_hbm.at[1, pl.ds(0, WIN)], db[sl], semi).wait()

    d0 = idx_fire(0, 0)
    d1 = idx_fire(1, 1)
    for d in (*d0, *d1):
        d.wait()
    pltpu.sync_copy(h_hbm.at[sb[0]], rb[0])
    pltpu.async_copy(h_hbm.at[sb[1]], rb[1], sem)
    idx_fire(2, 2)

    @pl.loop(0, 84, step=12)
    def _(vb):
        for u in range(12):
            v = vb + u
            sl = u & 3
            w = wid + NW * v

            @pl.when(w + 2 * NW < NWIN)
            def _():
                idx_wait((u + 2) & 3)
                pltpu.async_copy(h_hbm.at[sb[(u + 2) & 3]],
                                 rb[(u + 2) % 3], sem)

            @pl.when(w + 3 * NW < NWIN)
            def _():
                idx_fire(v + 3, (u + 3) & 3)

            @pl.when(w < NWIN)
            def _():
                pltpu.sync_copy(rb[u % 3], acc.at[db[sl]], add=True)

            @pl.when(w + NW < NWIN)
            def _():
                pltpu.make_async_copy(h_hbm.at[sb[(u + 1) & 3]],
                                      rb[(u + 1) % 3], sem).wait()

    plsc.subcore_barrier()

    @pl.when(s < NS - 1)
    def _():
        pltpu.sync_copy(acc.at[pl.ds(s * RPS, RPS)],
                        out_hbm.at[c, pl.ds(s * RPS, RPS)])

    @pl.when(s == NS - 1)
    def _():
        pltpu.sync_copy(acc.at[pl.ds((NS - 1) * RPS, RPSL)],
                        out_hbm.at[c, pl.ds((NS - 1) * RPS, RPSL)])


def _sc_agg(h, edge_index3, zeros):
    return _sc_agg_kernel(h, edge_index3, zeros)


def _dot_t(a, w):
    return lax.dot_general(a, w, (((1,), (1,)), ((), ())),
                           preferred_element_type=jnp.float32)


def _tc_init_body(x_ref, w_ref, b_ref, o_ref):
    o_ref[...] = _dot_t(x_ref[...], w_ref[...]) + b_ref[...]


def _tc_init(x, W_init, b2):
    return pl.pallas_call(
        _tc_init_body,
        out_shape=jax.ShapeDtypeStruct((N, D), jnp.float32),
    )(x, W_init, b2)


def _tc_layer_body(h_ref, p_ref, wr_ref, br_ref, wt_ref, g_ref, b_ref, o_ref):
    agg = p_ref[0, :N, :] + p_ref[1, :N, :]
    h = h_ref[...]
    t = h + _dot_t(agg, wr_ref[...]) + br_ref[...] + _dot_t(h, wt_ref[...])
    m = jnp.mean(t, axis=0, keepdims=True)
    v = jnp.mean((t - m) ** 2, axis=0, keepdims=True)
    t = (t - m) / jnp.sqrt(v + 1e-5) * g_ref[...] + b_ref[...]
    o_ref[...] = jnp.maximum(t, 0.0)


def _tc_layer(h, parts, Wr, br2, Wt, g2, b2):
    return pl.pallas_call(
        _tc_layer_body,
        out_shape=jax.ShapeDtypeStruct((N, D), jnp.float32),
    )(h, parts, Wr, br2, Wt, g2, b2)


def _tc_final_body(h_ref, p_ref, wr_ref, br_ref, wt_ref, batch_ref, o_ref):
    agg = p_ref[0, :N, :] + p_ref[1, :N, :]
    t = _dot_t(agg, wr_ref[...]) + br_ref[...] + _dot_t(h_ref[...], wt_ref[...])
    seg = lax.broadcasted_iota(jnp.int32, (G, N), 0)
    mask = (seg == batch_ref[...]).astype(jnp.float32)
    o_ref[...] = lax.dot_general(mask, t, (((1,), (0,)), ((), ())),
                                 preferred_element_type=jnp.float32)


def _tc_final(h, parts, Wr, br2, Wt, batch2):
    return pl.pallas_call(
        _tc_final_body,
        out_shape=jax.ShapeDtypeStruct((G, D), jnp.float32),
    )(h, parts, Wr, br2, Wt, batch2)


def kernel(x, edge_index, batch, W_init, b_init, W_rel, b_rel, W_root, gamma, beta):
    zeros = jnp.zeros((RPS, D), jnp.float32)
    batch2 = batch.reshape(1, N)
    e3 = edge_index
    h = _tc_init(x, W_init, b_init.reshape(1, D))
    for i in range(L - 1):
        parts = _sc_agg(h, e3, zeros)
        h = _tc_layer(h, parts, W_rel[i], b_rel[i].reshape(1, D),
                      W_root[i], gamma[i].reshape(1, D), beta[i].reshape(1, D))
    parts = _sc_agg(h, e3, zeros)
    return _tc_final(h, parts, W_rel[L - 1], b_rel[L - 1].reshape(1, D),
                     W_root[L - 1], batch2)
